# Initial kernel scaffold; baseline (speedup 1.0000x reference)
import jax
import jax.numpy as jnp
from jax.experimental import pallas as pl


def kernel(unit_duration_exec, source_duration_obs, unit_mask, sealed_mask, speech_commit_mask, unit_logstretch, basis_activation):
    raise NotImplementedError("write your pallas kernel here")


# speedup vs baseline: 4.7834x; 4.7834x over previous
import functools

import jax
import jax.numpy as jnp
from jax import lax
from jax.experimental import pallas as pl
from jax.experimental.pallas import tpu as pltpu
from jax.experimental.pallas import tpu_sc as plsc

B = 16
U = 2048
BUDGET_POS = 24.0
BUDGET_NEG = 24.0
UNROLL = 8

_mesh = plsc.VectorSubcoreMesh(core_axis_name="c", subcore_axis_name="s")


@functools.partial(
    pl.kernel,
    mesh=_mesh,
    out_type=jax.ShapeDtypeStruct((U * B,), jnp.float32),
    scratch_types=[
        pltpu.VMEM((U * B,), jnp.float32),
        pltpu.VMEM((U * B,), jnp.float32),
        pltpu.VMEM((U * B,), jnp.float32),
    ],
)
def _scan_kernel(d_hbm, e_hbm, out_hbm, d_v, e_v, o_v):
    wid = lax.axis_index("c") * 16 + lax.axis_index("s")

    @pl.when(wid == 0)
    def _():
        pltpu.sync_copy(d_hbm, d_v)
        pltpu.sync_copy(e_hbm, e_v)

        zero = jnp.zeros((B,), jnp.float32)

        def body(i, carry):
            c, off = carry
            base = i * (UNROLL * B)
            for j in range(UNROLL):
                idx = base + j * B
                e = e_v[pl.ds(idx, B)]
                du = d_v[pl.ds(idx, B)]
                a = e > 0.0
                total = jnp.maximum(0.0, du + c)
                f0 = lax.convert_element_type(
                    lax.convert_element_type(total, jnp.int32), jnp.float32)
                lower = jnp.maximum(1.0, (e - BUDGET_NEG) - off)
                upper = jnp.maximum(lower, (e + BUDGET_POS) - off)
                frames = jnp.minimum(jnp.maximum(f0, lower), upper)
                o_v[pl.ds(idx, B)] = jnp.where(a, frames, -e - 1.0)
                c = jnp.where(a, total - frames, c)
                off = jnp.where(a, off + frames - e, off)
            return c, off

        lax.fori_loop(0, U // UNROLL, body, (zero, zero), unroll=False)
        pltpu.sync_copy(o_v, out_hbm)


def kernel(unit_duration_exec, source_duration_obs, unit_mask, sealed_mask,
           speech_commit_mask, unit_logstretch=None, basis_activation=None):
    d = unit_duration_exec.astype(jnp.float32)
    s_f = source_duration_obs.astype(jnp.float32)
    src = jnp.maximum(0.0, jnp.round(s_f))
    anchor = jnp.maximum(1.0, src)
    cmask = unit_mask.astype(jnp.float32) * sealed_mask.astype(jnp.float32)
    committed = cmask > 0.5
    speech = speech_commit_mask.astype(jnp.float32) > 0.5
    act = committed & speech
    pinact = jnp.where(committed, src, 0.0)
    e = jnp.where(act, anchor, -pinact - 1.0).astype(jnp.float32)

    d_t = d.T.reshape(-1)
    e_t = e.T.reshape(-1)

    proj_t = _scan_kernel(d_t, e_t)
    proj = proj_t.reshape(U, B).T

    projected_prefix = proj * cmask
    return d + lax.stop_gradient(projected_prefix - d)
